# baseline (speedup 1.0000x reference)
import functools

import jax
import jax.numpy as jnp
from jax import lax
from jax.experimental import pallas as pl
from jax.experimental.pallas import tpu as pltpu
from jax.experimental.pallas import tpu_sc as plsc

B, N, C = 8, 20000, 91
S = 2000
NC, NS, L = 2, 16, 16
WORKERS = 25
CHUNK = S // WORKERS
W = 128


def _sc_pick(boxes2d, scores2d, bidx, lidx, nidx):
    mesh = plsc.VectorSubcoreMesh(core_axis_name="c", subcore_axis_name="s")

    @functools.partial(
        pl.kernel,
        mesh=mesh,
        compiler_params=pltpu.CompilerParams(needs_layout_passes=False),
        out_type=[jax.ShapeDtypeStruct((S,), jnp.float32) for _ in range(7)],
        scratch_types=[
            pltpu.VMEM((CHUNK,), jnp.int32),
            pltpu.VMEM((CHUNK,), jnp.int32),
            pltpu.VMEM((CHUNK,), jnp.int32),
            pltpu.VMEM((CHUNK,), jnp.int32),
            pltpu.VMEM((CHUNK,), jnp.int32),
            pltpu.VMEM((CHUNK,), jnp.int32),
            pltpu.VMEM((CHUNK,), jnp.int32),
            pltpu.VMEM((CHUNK, W), jnp.float32),
            pltpu.VMEM((CHUNK, W), jnp.float32),
            pltpu.VMEM((CHUNK,), jnp.float32),
            pltpu.VMEM((CHUNK,), jnp.float32),
            pltpu.VMEM((CHUNK,), jnp.float32),
            pltpu.VMEM((CHUNK,), jnp.float32),
            pltpu.VMEM((CHUNK,), jnp.float32),
            pltpu.VMEM((CHUNK,), jnp.float32),
            pltpu.VMEM((CHUNK,), jnp.float32),
            pltpu.SemaphoreType.DMA,
            pltpu.SemaphoreType.DMA,
        ],
    )
    def k(boxes_hbm, scores_hbm, bidx_hbm, lidx_hbm, nidx_hbm,
          o0_hbm, o1_hbm, o2_hbm, o3_hbm, o4_hbm, o5_hbm, o6_hbm,
          b_v, l_v, n_v, sr_v, scol_v, br_v, bcol_v,
          srows_v, brows_v, c0_v, c1_v, c2_v, c3_v, sc_v, bf_v, lf_v,
          sem_a, sem_b):
        wid = lax.axis_index("s") * NC + lax.axis_index("c")

        @pl.when(wid < WORKERS)
        def _():
            base = wid * CHUNK
            cps = [pltpu.async_copy(bidx_hbm.at[pl.ds(base, CHUNK)], b_v,
                                    sem_a),
                   pltpu.async_copy(lidx_hbm.at[pl.ds(base, CHUNK)], l_v,
                                    sem_a),
                   pltpu.async_copy(nidx_hbm.at[pl.ds(base, CHUNK)], n_v,
                                    sem_a)]
            for cp in cps:
                cp.wait()

            for j in range(CHUNK // L):
                sl = pl.ds(j * L, L)
                row = b_v[sl] * N + n_v[sl]
                fs = row * C + l_v[sl]
                fb = row * 4
                sr_v[sl] = fs >> 7
                scol_v[sl] = fs & (W - 1)
                br_v[sl] = fb >> 7
                bcol_v[sl] = fb & (W - 1)
                bf_v[sl] = b_v[sl].astype(jnp.float32)
                lf_v[sl] = l_v[sl].astype(jnp.float32)

            fetch_cps = []
            for j in range(CHUNK // L):
                srvec = sr_v[pl.ds(j * L, L)]
                brvec = br_v[pl.ds(j * L, L)]
                for kk in range(L):
                    i = j * L + kk
                    fetch_cps.append(pltpu.async_copy(
                        scores_hbm.at[pl.ds(srvec[kk], 1)],
                        srows_v.at[pl.ds(i, 1)], sem_a))
                    fetch_cps.append(pltpu.async_copy(
                        boxes_hbm.at[pl.ds(brvec[kk], 1)],
                        brows_v.at[pl.ds(i, 1)], sem_b))
            for cp in fetch_cps:
                cp.wait()

            iota = lax.iota(jnp.int32, L)
            for j in range(CHUNK // L):
                sl = pl.ds(j * L, L)
                rid = iota + (j * L)
                sc_v[sl] = plsc.load_gather(srows_v, [rid, scol_v[sl]])
                bc = bcol_v[sl]
                c0_v[sl] = plsc.load_gather(brows_v, [rid, bc])
                c1_v[sl] = plsc.load_gather(brows_v, [rid, bc + 1])
                c2_v[sl] = plsc.load_gather(brows_v, [rid, bc + 2])
                c3_v[sl] = plsc.load_gather(brows_v, [rid, bc + 3])

            outs = [(bf_v, o0_hbm), (c0_v, o1_hbm), (c1_v, o2_hbm),
                    (c2_v, o3_hbm), (c3_v, o4_hbm), (sc_v, o5_hbm),
                    (lf_v, o6_hbm)]
            cps = [pltpu.async_copy(src, dst.at[pl.ds(base, CHUNK)], sem_b)
                   for src, dst in outs]
            for cp in cps:
                cp.wait()

    return k(boxes2d, scores2d, bidx, lidx, nidx)


def kernel(pred_boxes, pred_scores, selected_indexes):
    sel = selected_indexes.astype(jnp.int32)
    boxes2d = pred_boxes.reshape(B * N * 4 // W, W)
    scores2d = pred_scores.reshape(B * N * C // W, W)
    cols = _sc_pick(boxes2d, scores2d, sel[:, 0], sel[:, 1], sel[:, 2])
    return jnp.stack(cols, axis=1)


# speedup vs baseline: 2.8400x; 2.8400x over previous
import functools

import jax
import jax.numpy as jnp
from jax import lax
from jax.experimental import pallas as pl
from jax.experimental.pallas import tpu as pltpu
from jax.experimental.pallas import tpu_sc as plsc

B, N, C = 8, 20000, 91
S = 2000
NC, NS, L = 2, 16, 16
WORKERS = 25
CHUNK = S // WORKERS


def _sc_pick(boxes3d, scores3d, bidx, lidx, nidx):
    mesh = plsc.VectorSubcoreMesh(core_axis_name="c", subcore_axis_name="s")

    @functools.partial(
        pl.kernel,
        mesh=mesh,
        compiler_params=pltpu.CompilerParams(needs_layout_passes=False),
        out_type=[jax.ShapeDtypeStruct((S,), jnp.float32) for _ in range(7)],
        scratch_types=[
            pltpu.VMEM((CHUNK,), jnp.int32),
            pltpu.VMEM((CHUNK,), jnp.int32),
            pltpu.VMEM((CHUNK,), jnp.int32),
            pltpu.VMEM((CHUNK, 4), jnp.float32),
            pltpu.VMEM((CHUNK,), jnp.float32),
            pltpu.VMEM((CHUNK,), jnp.float32),
            pltpu.VMEM((CHUNK,), jnp.float32),
            pltpu.VMEM((CHUNK,), jnp.float32),
            pltpu.VMEM((CHUNK, C), jnp.float32),
            pltpu.VMEM((CHUNK,), jnp.float32),
            pltpu.VMEM((CHUNK,), jnp.float32),
            pltpu.VMEM((CHUNK,), jnp.float32),
            pltpu.SemaphoreType.DMA,
            pltpu.SemaphoreType.DMA,
        ],
    )
    def k(boxes_hbm, scores_hbm, bidx_hbm, lidx_hbm, nidx_hbm,
          o0_hbm, o1_hbm, o2_hbm, o3_hbm, o4_hbm, o5_hbm, o6_hbm,
          b_v, l_v, n_v, boxrows_v, c0_v, c1_v, c2_v, c3_v,
          rows_v, sc_v, bf_v, lf_v, sem_a, sem_b):
        wid = lax.axis_index("s") * NC + lax.axis_index("c")

        @pl.when(wid < WORKERS)
        def _():
            base = wid * CHUNK
            cps = [pltpu.async_copy(bidx_hbm.at[pl.ds(base, CHUNK)], b_v,
                                    sem_a),
                   pltpu.async_copy(lidx_hbm.at[pl.ds(base, CHUNK)], l_v,
                                    sem_a),
                   pltpu.async_copy(nidx_hbm.at[pl.ds(base, CHUNK)], n_v,
                                    sem_a)]
            for cp in cps:
                cp.wait()

            row_cps = []
            for j in range(CHUNK // L):
                bvec = b_v[pl.ds(j * L, L)]
                nvec = n_v[pl.ds(j * L, L)]
                for kk in range(L):
                    i = j * L + kk
                    row_cps.append(pltpu.async_copy(
                        boxes_hbm.at[bvec[kk], pl.ds(nvec[kk], 1)],
                        boxrows_v.at[pl.ds(i, 1)], sem_b))
                    row_cps.append(pltpu.async_copy(
                        scores_hbm.at[bvec[kk], pl.ds(nvec[kk], 1)],
                        rows_v.at[pl.ds(i, 1)], sem_a))

            for j in range(CHUNK // L):
                sl = pl.ds(j * L, L)
                bf_v[sl] = b_v[sl].astype(jnp.float32)
                lf_v[sl] = l_v[sl].astype(jnp.float32)

            for cp in row_cps:
                cp.wait()

            iota = lax.iota(jnp.int32, L)
            for j in range(CHUNK // L):
                sl = pl.ds(j * L, L)
                rid = iota + (j * L)
                sc_v[sl] = plsc.load_gather(rows_v, [rid, l_v[sl]])
                for c, cv in enumerate((c0_v, c1_v, c2_v, c3_v)):
                    cv[sl] = plsc.load_gather(
                        boxrows_v, [rid, jnp.full((L,), c, jnp.int32)])

            outs = [(bf_v, o0_hbm), (c0_v, o1_hbm), (c1_v, o2_hbm),
                    (c2_v, o3_hbm), (c3_v, o4_hbm), (sc_v, o5_hbm),
                    (lf_v, o6_hbm)]
            cps = [pltpu.async_copy(src, dst.at[pl.ds(base, CHUNK)], sem_b)
                   for src, dst in outs]
            for cp in cps:
                cp.wait()

    return k(boxes3d, scores3d, bidx, lidx, nidx)


def kernel(pred_boxes, pred_scores, selected_indexes):
    sel = selected_indexes.astype(jnp.int32)
    cols = _sc_pick(pred_boxes, pred_scores, sel[:, 0], sel[:, 1], sel[:, 2])
    return jnp.stack(cols, axis=1)
